# Initial kernel scaffold; baseline (speedup 1.0000x reference)
import jax
import jax.numpy as jnp
from jax.experimental import pallas as pl


def kernel(feature, edge_index, W, b):
    raise NotImplementedError("write your pallas kernel here")


# speedup vs baseline: 7.1588x; 7.1588x over previous
import functools

import jax
import jax.numpy as jnp
from jax import lax
from jax.experimental import pallas as pl
from jax.experimental.pallas import tpu as pltpu
from jax.experimental.pallas import tpu_sc as plsc

N_NODES = 10000
N_EDGES = 320000
D = 128

NC = 2
NS = 16
NW = NC * NS
EPW = N_EDGES // NW
CH = 80
NCHUNK = EPW // CH
IBLK = 25
NBLK = NCHUNK // IBLK
NPAD = 10240
SLICE = NPAD // NS
SUB = CH


def _tables_body(feat_ref, w_ref, b_ref, a_ref, bt_ref):
    w1 = w_ref[:, :D]
    w2 = w_ref[:, D:]
    f = feat_ref[...]
    dn = (((1,), (1,)), ((), ()))
    a_ref[...] = lax.dot_general(f, w1 - w2, dn,
                                 preferred_element_type=jnp.float32) + b_ref[...]
    bt_ref[...] = lax.dot_general(f, w2, dn,
                                  preferred_element_type=jnp.float32)


def _make_tables(feature, W, b):
    return pl.pallas_call(
        _tables_body,
        out_shape=(
            jax.ShapeDtypeStruct((N_NODES, D), jnp.float32),
            jax.ShapeDtypeStruct((N_NODES, D), jnp.float32),
        ),
    )(feature, W, b.reshape(1, D))


def _edge_body(a_hbm, b_hbm, src_hbm, dst_hbm, psum_hbm, pcnt_hbm,
               idx_src, idx_dst, buf_a, buf_b, hist,
               acc, sem_a, sem_b):
    c = lax.axis_index("c")
    s = lax.axis_index("s")
    w = c * NS + s

    zeros16 = jnp.zeros((16,), jnp.float32)

    def _fill_buf(i, _):
        for j in range(D // 16):
            buf_a[i, pl.ds(j * 16, 16)] = zeros16
        return 0
    lax.fori_loop(0, CH, _fill_buf, 0)

    def _fill_hist(i, _):
        hist[pl.ds(i * 16, 16)] = zeros16
        return 0
    lax.fori_loop(0, NPAD // 16, _fill_hist, 0)

    base = s * SLICE
    for k in range(SLICE // SUB):
        pltpu.sync_copy(buf_a, acc.at[pl.ds(base + k * SUB, SUB)])
    plsc.subcore_barrier()

    def _block(bi, _):
        pltpu.sync_copy(src_hbm.at[w, bi], idx_src)
        pltpu.sync_copy(dst_hbm.at[w, bi], idx_dst)

        def _chunk(ci, _):
            idxd = idx_dst.at[ci]
            idxs = idx_src.at[ci]
            cp_a = pltpu.async_copy(a_hbm.at[idxd], buf_a, sem_a)
            cp_b = pltpu.async_copy(b_hbm.at[idxs], buf_b, sem_b)
            cp_a.wait()
            cp_b.wait()

            def _row(i, _):
                for j in range(D // 16):
                    sl = pl.ds(j * 16, 16)
                    buf_a[i, sl] = jnp.maximum(buf_a[i, sl] + buf_b[i, sl],
                                               0.0)
                return 0
            lax.fori_loop(0, CH, _row, 0)

            e0 = jnp.where(lax.iota(jnp.int32, 16) == 0, 1.0, 0.0)

            def _cnt(k, _):
                idxv = idx_dst[ci, pl.ds(k * 16, 16)]
                for l in range(16):
                    hsl = pl.ds(idxv[l], 16)
                    hist[hsl] = hist[hsl] + e0
                return 0
            lax.fori_loop(0, CH // 16, _cnt, 0)

            pltpu.sync_copy(buf_a, acc.at[idxd], add=True)
            return 0
        lax.fori_loop(0, IBLK, _chunk, 0)
        return 0
    lax.fori_loop(0, NBLK, _block, 0)

    plsc.subcore_barrier()

    for k in range(SLICE // SUB):
        off = base + k * SUB
        pltpu.sync_copy(acc.at[pl.ds(off, SUB)], buf_a)
        pltpu.sync_copy(buf_a, psum_hbm.at[c, pl.ds(off, SUB)])
    pltpu.sync_copy(hist, pcnt_hbm.at[w])


@functools.partial(
    pl.kernel,
    out_type=(
        jax.ShapeDtypeStruct((NC, NPAD, D), jnp.float32),
        jax.ShapeDtypeStruct((NW, NPAD), jnp.float32),
    ),
    mesh=plsc.VectorSubcoreMesh(core_axis_name="c", subcore_axis_name="s"),
    scratch_types=[
        pltpu.VMEM((IBLK, CH), jnp.int32),
        pltpu.VMEM((IBLK, CH), jnp.int32),
        pltpu.VMEM((CH, D), jnp.float32),
        pltpu.VMEM((CH, D), jnp.float32),
        pltpu.VMEM((NPAD,), jnp.float32),
        pltpu.VMEM_SHARED((NPAD, D), jnp.float32),
        pltpu.SemaphoreType.DMA,
        pltpu.SemaphoreType.DMA,
    ],
)
def _edge_kernel(a_hbm, b_hbm, src_hbm, dst_hbm, psum_hbm, pcnt_hbm,
                 idx_src, idx_dst, buf_a, buf_b, hist,
                 acc, sem_a, sem_b):
    _edge_body(a_hbm, b_hbm, src_hbm, dst_hbm, psum_hbm, pcnt_hbm,
               idx_src, idx_dst, buf_a, buf_b, hist,
               acc, sem_a, sem_b)


def _final_body(psum_ref, pcnt_ref, out_ref):
    tot = psum_ref[0, :N_NODES] + psum_ref[1, :N_NODES]
    cnt = jnp.sum(pcnt_ref[...], axis=0)
    cntcol = cnt[:N_NODES].reshape(N_NODES, 1)
    out_ref[...] = tot / jnp.maximum(cntcol, 1.0)


def _finalize(psum, pcnt):
    return pl.pallas_call(
        _final_body,
        out_shape=jax.ShapeDtypeStruct((N_NODES, D), jnp.float32),
    )(psum, pcnt)


def kernel(feature, edge_index, W, b):
    a_tab, b_tab = _make_tables(feature, W, b)
    src4 = edge_index[0].reshape(NW, NBLK, IBLK, CH)
    dst4 = edge_index[1].reshape(NW, NBLK, IBLK, CH)
    psum, pcnt = _edge_kernel(a_tab, b_tab, src4, dst4)
    return _finalize(psum, pcnt)
